# Initial kernel scaffold; baseline (speedup 1.0000x reference)
import jax
import jax.numpy as jnp
from jax.experimental import pallas as pl


def kernel(features):
    raise NotImplementedError("write your pallas kernel here")


# speedup vs baseline: 2.8225x; 2.8225x over previous
import functools

import jax
import jax.numpy as jnp
import numpy as np
from jax import lax
from jax.experimental import pallas as pl
from jax.experimental.pallas import tpu as pltpu
from jax.experimental.pallas import tpu_sc as plsc

_DROP_THRESHOLD = 0.25

_NC = 2
_NS = 16
_NW = _NC * _NS
_L = 16


def _tf2x32(k1, k2, x1, x2):
    rot0 = (13, 15, 26, 6)
    rot1 = (17, 29, 16, 24)
    ks = (np.uint32(k1), np.uint32(k2),
          np.uint32(k1) ^ np.uint32(k2) ^ np.uint32(0x1BD11BDA))
    x = [(x1 + ks[0]).astype(np.uint32), (x2 + ks[1]).astype(np.uint32)]

    def rnd(x, r):
        a = (x[0] + x[1]).astype(np.uint32)
        b = ((x[1] << np.uint32(r)) | (x[1] >> np.uint32(32 - r))).astype(np.uint32)
        return [a, a ^ b]

    sched = [(rot0, 1, 2, 1), (rot1, 2, 0, 2), (rot0, 0, 1, 3),
             (rot1, 1, 2, 4), (rot0, 2, 0, 5)]
    for rots, i0, i1, c in sched:
        for r in rots:
            x = rnd(x, r)
        x = [(x[0] + ks[i0]).astype(np.uint32),
             (x[1] + ks[i1] + np.uint32(c)).astype(np.uint32)]
    return x


@functools.lru_cache(maxsize=None)
def _drop_index(hw: int, drop_num: int):
    def split2(k1, k2):
        b1, b2 = _tf2x32(k1, k2, np.zeros(2, np.uint32),
                         np.arange(2, dtype=np.uint32))
        return (b1[0], b2[0]), (b1[1], b2[1])

    k = (np.uint32(0), np.uint32(1234))
    x = np.arange(hw, dtype=np.int32)
    num_rounds = int(np.ceil(3 * np.log(max(1, hw))
                             / np.log(np.iinfo(np.uint32).max)))
    for _ in range(num_rounds):
        k, sub = split2(*k)
        b1, b2 = _tf2x32(sub[0], sub[1], np.zeros(hw, np.uint32),
                         np.arange(hw, dtype=np.uint32))
        x = x[np.argsort(b1 ^ b2, kind="stable")]
    return x[:drop_num].astype(np.int32)


def _mask_sc(index, ones_seg, hw: int, drop_num: int):
    seg = hw // _NW
    n_vec = drop_num // _L

    mesh = plsc.VectorSubcoreMesh(core_axis_name="c", subcore_axis_name="s")

    @functools.partial(
        pl.kernel,
        mesh=mesh,
        out_type=jax.ShapeDtypeStruct((hw,), jnp.float32),
        scratch_types=[
            pltpu.VMEM((drop_num,), jnp.int32),
            pltpu.VMEM((seg,), jnp.float32),
        ],
        compiler_params=pltpu.CompilerParams(needs_layout_passes=False),
    )
    def build(idx_hbm, ones_hbm, out_hbm, idx_v, seg_v):
        wid = lax.axis_index("s") * _NC + lax.axis_index("c")
        base = wid * seg
        pltpu.sync_copy(idx_hbm, idx_v)
        pltpu.sync_copy(ones_hbm, seg_v)
        zeros16 = jnp.zeros((_L,), jnp.float32)

        def scat_body(i, carry):
            idx16 = idx_v[pl.ds(pl.multiple_of(i * _L, _L), _L)]
            local = idx16 - base
            inb = (local >= 0) & (local < seg)
            safe = jnp.clip(local, 0, seg - 1)
            plsc.store_scatter(seg_v, [safe], zeros16, mask=inb)
            return carry

        lax.fori_loop(0, n_vec, scat_body, 0)
        pltpu.sync_copy(seg_v, out_hbm.at[pl.ds(base, seg)])

    return build(index, ones_seg)


def _mul_body(f_ref, m_ref, o_ref):
    o_ref[...] = f_ref[...] * m_ref[...]


def _apply_mask(feats3, mask3, bk: int):
    n, h, w = feats3.shape
    return pl.pallas_call(
        _mul_body,
        grid=(n // bk,),
        in_specs=[
            pl.BlockSpec((bk, h, w), lambda i: (i, 0, 0)),
            pl.BlockSpec((1, h, w), lambda i: (0, 0, 0)),
        ],
        out_specs=pl.BlockSpec((bk, h, w), lambda i: (i, 0, 0)),
        out_shape=jax.ShapeDtypeStruct((n, h, w), jnp.float32),
        compiler_params=pltpu.CompilerParams(
            dimension_semantics=("arbitrary",),
        ),
    )(feats3, mask3)


def kernel(features):
    b, c, h, w = features.shape
    hw = h * w
    drop_num = int(_DROP_THRESHOLD * hw)
    index = jnp.asarray(_drop_index(hw, drop_num))
    ones_seg = jnp.ones((hw // _NW,), jnp.float32)
    mask = _mask_sc(index, ones_seg, hw, drop_num)
    feats3 = features.reshape(b * c, h, w)
    out3 = _apply_mask(feats3, mask.reshape(1, h, w), bk=4)
    return out3.reshape(b, c, h, w)


# speedup vs baseline: 3.0710x; 1.0881x over previous
import functools

import jax
import jax.numpy as jnp
import numpy as np
from jax import lax
from jax.experimental import pallas as pl
from jax.experimental.pallas import tpu as pltpu
from jax.experimental.pallas import tpu_sc as plsc

_DROP_THRESHOLD = 0.25

_NC = 2
_NS = 16
_NW = _NC * _NS
_L = 16


def _tf2x32(k1, k2, x1, x2):
    rot0 = (13, 15, 26, 6)
    rot1 = (17, 29, 16, 24)
    ks = (np.uint32(k1), np.uint32(k2),
          np.uint32(k1) ^ np.uint32(k2) ^ np.uint32(0x1BD11BDA))
    x = [(x1 + ks[0]).astype(np.uint32), (x2 + ks[1]).astype(np.uint32)]

    def rnd(x, r):
        a = (x[0] + x[1]).astype(np.uint32)
        b = ((x[1] << np.uint32(r)) | (x[1] >> np.uint32(32 - r))).astype(np.uint32)
        return [a, a ^ b]

    sched = [(rot0, 1, 2, 1), (rot1, 2, 0, 2), (rot0, 0, 1, 3),
             (rot1, 1, 2, 4), (rot0, 2, 0, 5)]
    for rots, i0, i1, c in sched:
        for r in rots:
            x = rnd(x, r)
        x = [(x[0] + ks[i0]).astype(np.uint32),
             (x[1] + ks[i1] + np.uint32(c)).astype(np.uint32)]
    return x


@functools.lru_cache(maxsize=None)
def _drop_index(hw: int, drop_num: int):
    def split2(k1, k2):
        b1, b2 = _tf2x32(k1, k2, np.zeros(2, np.uint32),
                         np.arange(2, dtype=np.uint32))
        return (b1[0], b2[0]), (b1[1], b2[1])

    k = (np.uint32(0), np.uint32(1234))
    x = np.arange(hw, dtype=np.int32)
    num_rounds = int(np.ceil(3 * np.log(max(1, hw))
                             / np.log(np.iinfo(np.uint32).max)))
    for _ in range(num_rounds):
        k, sub = split2(*k)
        b1, b2 = _tf2x32(sub[0], sub[1], np.zeros(hw, np.uint32),
                         np.arange(hw, dtype=np.uint32))
        x = x[np.argsort(b1 ^ b2, kind="stable")]
    return x[:drop_num].astype(np.int32)


@functools.lru_cache(maxsize=None)
def _bucketed_index(hw: int, drop_num: int):
    idx = _drop_index(hw, drop_num)
    seg = hw // _NW
    owner = idx // seg
    width = int(np.bincount(owner, minlength=_NW).max())
    width = -(-width // _L) * _L
    buckets = np.full((_NW, width), seg, dtype=np.int32)
    for t in range(_NW):
        local = idx[owner == t] - t * seg
        buckets[t, : local.size] = local
    return buckets, width


def _mask_sc(buckets, ones_seg, hw: int, width: int):
    seg = hw // _NW

    mesh = plsc.VectorSubcoreMesh(core_axis_name="c", subcore_axis_name="s")

    @functools.partial(
        pl.kernel,
        mesh=mesh,
        out_type=jax.ShapeDtypeStruct((hw,), jnp.float32),
        scratch_types=[
            pltpu.VMEM((width,), jnp.int32),
            pltpu.VMEM((seg + _L,), jnp.float32),
        ],
        compiler_params=pltpu.CompilerParams(needs_layout_passes=False),
    )
    def build(idx_hbm, ones_hbm, out_hbm, idx_v, seg_v):
        wid = lax.axis_index("s") * _NC + lax.axis_index("c")
        base = wid * seg
        pltpu.sync_copy(idx_hbm.at[wid], idx_v)
        pltpu.sync_copy(ones_hbm, seg_v.at[pl.ds(0, seg)])
        zeros16 = jnp.zeros((_L,), jnp.float32)

        def scat_body(i, carry):
            idx16 = idx_v[pl.ds(pl.multiple_of(i * _L, _L), _L)]
            plsc.store_scatter(seg_v, [idx16], zeros16)
            return carry

        lax.fori_loop(0, width // _L, scat_body, 0)
        pltpu.sync_copy(seg_v.at[pl.ds(0, seg)], out_hbm.at[pl.ds(base, seg)])

    return build(buckets, ones_seg)


def _mul_body(f_ref, m_ref, o_ref):
    o_ref[...] = f_ref[...] * m_ref[...]


def _apply_mask(feats3, mask3, bk: int):
    n, h, w = feats3.shape
    return pl.pallas_call(
        _mul_body,
        grid=(n // bk,),
        in_specs=[
            pl.BlockSpec((bk, h, w), lambda i: (i, 0, 0)),
            pl.BlockSpec((1, h, w), lambda i: (0, 0, 0)),
        ],
        out_specs=pl.BlockSpec((bk, h, w), lambda i: (i, 0, 0)),
        out_shape=jax.ShapeDtypeStruct((n, h, w), jnp.float32),
        compiler_params=pltpu.CompilerParams(
            dimension_semantics=("arbitrary",),
        ),
    )(feats3, mask3)


def kernel(features):
    b, c, h, w = features.shape
    hw = h * w
    drop_num = int(_DROP_THRESHOLD * hw)
    buckets, width = _bucketed_index(hw, drop_num)
    ones_seg = jnp.ones((hw // _NW,), jnp.float32)
    mask = _mask_sc(jnp.asarray(buckets), ones_seg, hw, width)
    feats3 = features.reshape(b * c, h, w)
    out3 = _apply_mask(feats3, mask.reshape(1, h, w), bk=8)
    return out3.reshape(b, c, h, w)


# speedup vs baseline: 3.0787x; 1.0025x over previous
import functools

import jax
import jax.numpy as jnp
import numpy as np
from jax import lax
from jax.experimental import pallas as pl
from jax.experimental.pallas import tpu as pltpu
from jax.experimental.pallas import tpu_sc as plsc

_DROP_THRESHOLD = 0.25

_NC = 2
_NS = 16
_NW = _NC * _NS
_L = 16


def _tf2x32(k1, k2, x1, x2):
    rot0 = (13, 15, 26, 6)
    rot1 = (17, 29, 16, 24)
    ks = (np.uint32(k1), np.uint32(k2),
          np.uint32(k1) ^ np.uint32(k2) ^ np.uint32(0x1BD11BDA))
    x = [(x1 + ks[0]).astype(np.uint32), (x2 + ks[1]).astype(np.uint32)]

    def rnd(x, r):
        a = (x[0] + x[1]).astype(np.uint32)
        b = ((x[1] << np.uint32(r)) | (x[1] >> np.uint32(32 - r))).astype(np.uint32)
        return [a, a ^ b]

    sched = [(rot0, 1, 2, 1), (rot1, 2, 0, 2), (rot0, 0, 1, 3),
             (rot1, 1, 2, 4), (rot0, 2, 0, 5)]
    for rots, i0, i1, c in sched:
        for r in rots:
            x = rnd(x, r)
        x = [(x[0] + ks[i0]).astype(np.uint32),
             (x[1] + ks[i1] + np.uint32(c)).astype(np.uint32)]
    return x


@functools.lru_cache(maxsize=None)
def _drop_index(hw: int, drop_num: int):
    def split2(k1, k2):
        b1, b2 = _tf2x32(k1, k2, np.zeros(2, np.uint32),
                         np.arange(2, dtype=np.uint32))
        return (b1[0], b2[0]), (b1[1], b2[1])

    k = (np.uint32(0), np.uint32(1234))
    x = np.arange(hw, dtype=np.int32)
    num_rounds = int(np.ceil(3 * np.log(max(1, hw))
                             / np.log(np.iinfo(np.uint32).max)))
    for _ in range(num_rounds):
        k, sub = split2(*k)
        b1, b2 = _tf2x32(sub[0], sub[1], np.zeros(hw, np.uint32),
                         np.arange(hw, dtype=np.uint32))
        x = x[np.argsort(b1 ^ b2, kind="stable")]
    return x[:drop_num].astype(np.int32)


@functools.lru_cache(maxsize=None)
def _bucketed_index(hw: int, drop_num: int):
    idx = _drop_index(hw, drop_num)
    seg = hw // _NW
    owner = idx // seg
    width = int(np.bincount(owner, minlength=_NW).max())
    width = -(-width // _L) * _L
    buckets = np.full((_NW, width), seg, dtype=np.int32)
    for t in range(_NW):
        local = idx[owner == t] - t * seg
        buckets[t, : local.size] = local
    return buckets, width


def _mask_sc(buckets, ones_seg, hw: int, width: int):
    seg = hw // _NW

    mesh = plsc.VectorSubcoreMesh(core_axis_name="c", subcore_axis_name="s")

    @functools.partial(
        pl.kernel,
        mesh=mesh,
        out_type=jax.ShapeDtypeStruct((hw,), jnp.float32),
        scratch_types=[
            pltpu.VMEM((width,), jnp.int32),
            pltpu.VMEM((seg + _L,), jnp.float32),
        ],
        compiler_params=pltpu.CompilerParams(needs_layout_passes=False),
    )
    def build(idx_hbm, ones_hbm, out_hbm, idx_v, seg_v):
        wid = lax.axis_index("s") * _NC + lax.axis_index("c")
        base = wid * seg
        pltpu.sync_copy(idx_hbm.at[wid], idx_v)
        pltpu.sync_copy(ones_hbm, seg_v.at[pl.ds(0, seg)])
        zeros16 = jnp.zeros((_L,), jnp.float32)

        def scat_body(i, carry):
            idx16 = idx_v[pl.ds(pl.multiple_of(i * _L, _L), _L)]
            plsc.store_scatter(seg_v, [idx16], zeros16)
            return carry

        lax.fori_loop(0, width // _L, scat_body, 0)
        pltpu.sync_copy(seg_v.at[pl.ds(0, seg)], out_hbm.at[pl.ds(base, seg)])

    return build(buckets, ones_seg)


def _mul_body(f_ref, m_ref, o_ref):
    o_ref[...] = f_ref[...] * m_ref[...]


def _apply_mask(feats3, mask3, bk: int):
    n, h, w = feats3.shape
    return pl.pallas_call(
        _mul_body,
        grid=(n // bk,),
        in_specs=[
            pl.BlockSpec((bk, h, w), lambda i: (i, 0, 0)),
            pl.BlockSpec((1, h, w), lambda i: (0, 0, 0)),
        ],
        out_specs=pl.BlockSpec((bk, h, w), lambda i: (i, 0, 0)),
        out_shape=jax.ShapeDtypeStruct((n, h, w), jnp.float32),
        compiler_params=pltpu.CompilerParams(
            dimension_semantics=("arbitrary",),
        ),
    )(feats3, mask3)


def kernel(features):
    b, c, h, w = features.shape
    hw = h * w
    drop_num = int(_DROP_THRESHOLD * hw)
    buckets, width = _bucketed_index(hw, drop_num)
    ones_seg = jnp.ones((hw // _NW,), jnp.float32)
    mask = _mask_sc(jnp.asarray(buckets), ones_seg, hw, width)
    feats3 = features.reshape(b * c, h, w)
    out3 = _apply_mask(feats3, mask.reshape(1, h, w), bk=12)
    return out3.reshape(b, c, h, w)
